# baseline (speedup 1.0000x reference)
import functools

import jax
import jax.numpy as jnp
from jax import lax
from jax.experimental import pallas as pl
from jax.experimental.pallas import tpu as pltpu
from jax.experimental.pallas import tpu_sc as plsc

_EMBED_DIM = 32
_NUM_EMB = 8192
_BETA = 0.25


def _conv(x, w, b, stride, pad):
    y = lax.conv_general_dilated(
        x, w, (stride, stride), ((pad, pad), (pad, pad)),
        dimension_numbers=('NCHW', 'OIHW', 'NCHW'))
    return y + b[None, :, None, None]


def _convT(x, w, b):
    y = lax.conv_transpose(x, w, (2, 2), 'VALID',
                           dimension_numbers=('NCHW', 'OIHW', 'NCHW'))
    return y + b[None, :, None, None]


def _block(x, p):
    x = jax.nn.relu(_conv(x, p['w1'], p['b1'], 1, 1))
    x = jax.nn.relu(_conv(x, p['w2'], p['b2'], 1, 1))
    return x


def _maxpool(x):
    return lax.reduce_window(x, -jnp.inf, lax.max,
                             (1, 1, 2, 2), (1, 1, 2, 2), 'VALID')


def _argmin_kernel(n_tok, z_ref, cb_ref, idx_ref, loss_ref):
    z = z_ref[...]
    cb = cb_ref[...]
    pad = z.shape[0]
    zn = jnp.sum(z * z, axis=1, keepdims=True)
    cn = jnp.sum(cb * cb, axis=1)[None, :]
    dot = lax.dot_general(z, cb, (((1,), (1,)), ((), ())),
                          preferred_element_type=jnp.float32)
    d2 = (zn + cn) - 2.0 * dot
    d2 = jnp.maximum(d2, 0.0)
    mn = jnp.min(d2, axis=1, keepdims=True)
    lane = lax.broadcasted_iota(jnp.int32, d2.shape, 1)
    idx = jnp.min(jnp.where(d2 == mn, lane, _NUM_EMB), axis=1)
    idx_ref[...] = idx.reshape(1, pad)
    mask = (lax.broadcasted_iota(jnp.int32, (pad, 1), 0) < n_tok
            ).astype(jnp.float32)
    m = jnp.sum(mn * mask, axis=0, keepdims=True) / (n_tok * _EMBED_DIM)
    loss_ref[...] = m + _BETA * m


_SC_CORES = 2
_SC_SUBCORES = 16
_NW = _SC_CORES * _SC_SUBCORES
_BPAD = 8 * _NW
_BPW = _BPAD // _NW


def _sc_gather_body(cb_hbm, idx_hbm, out_hbm, idx_v, rows_v, sem):
    wid = lax.axis_index("s") * _SC_CORES + lax.axis_index("c")
    base = wid * _BPW
    pltpu.sync_copy(idx_hbm.at[pl.ds(base, _BPW)], idx_v)
    pltpu.async_copy(cb_hbm.at[idx_v], rows_v, sem).wait()
    pltpu.sync_copy(rows_v, out_hbm.at[pl.ds(base, _BPW)])


_sc_gather = functools.partial(
    pl.kernel,
    out_type=jax.ShapeDtypeStruct((_BPAD, _EMBED_DIM), jnp.float32),
    mesh=plsc.VectorSubcoreMesh(core_axis_name="c", subcore_axis_name="s",
                                num_cores=_SC_CORES,
                                num_subcores=_SC_SUBCORES),
    compiler_params=pltpu.CompilerParams(use_tc_tiling_on_sc=False),
    scratch_types=[
        pltpu.VMEM((_BPW,), jnp.int32),
        pltpu.VMEM((_BPW, _EMBED_DIM), jnp.float32),
        pltpu.SemaphoreType.DMA,
    ],
)(_sc_gather_body)


def _vq(z_flat, cb):
    n_tok = z_flat.shape[0]
    padded = max(8, -(-n_tok // 8) * 8)
    zp = jnp.pad(z_flat, ((0, padded - n_tok), (0, 0)))
    idx, loss = pl.pallas_call(
        functools.partial(_argmin_kernel, n_tok),
        out_shape=(jax.ShapeDtypeStruct((1, padded), jnp.int32),
                   jax.ShapeDtypeStruct((1, 1), jnp.float32)),
    )(zp, cb)
    idx_pad = jnp.pad(idx.reshape(padded), (0, _BPAD - padded))
    zq_pad = _sc_gather(cb, idx_pad)
    return zq_pad[:n_tok], loss[0, 0]


def kernel(x, params):
    h = x
    for p in params['enc']:
        h = _maxpool(_block(h, p))
    z_e = _conv(h, params['pre_w'], params['pre_b'], 1, 0)
    B, C, H, W = z_e.shape
    z_e_flat = jnp.transpose(z_e, (0, 2, 3, 1)).reshape(B * H * W, C)
    z_q, q_loss = _vq(z_e_flat, params['codebook'])
    latent = jnp.transpose(z_q.reshape(B, H, W, C), (0, 3, 1, 2))
    z = _conv(latent, params['post_w'], params['post_b'], 1, 0)
    for p in params['dec']:
        z = _block(_convT(z, p['wt'], p['bt']), p)
    x_reconst = jnp.tanh(_convT(z, params['dec_final_w'],
                                params['dec_final_b']))
    return (x_reconst, latent, q_loss)


# speedup vs baseline: 1.0227x; 1.0227x over previous
import functools

import jax
import jax.numpy as jnp
from jax import lax
from jax.experimental import pallas as pl
from jax.experimental.pallas import tpu as pltpu
from jax.experimental.pallas import tpu_sc as plsc

_EMBED_DIM = 32
_NUM_EMB = 8192
_BETA = 0.25


def _conv(x, w, b, stride, pad):
    y = lax.conv_general_dilated(
        x, w, (stride, stride), ((pad, pad), (pad, pad)),
        dimension_numbers=('NCHW', 'OIHW', 'NCHW'))
    return y + b[None, :, None, None]


def _convT(x, w, b):
    y = lax.conv_transpose(x, w, (2, 2), 'VALID',
                           dimension_numbers=('NCHW', 'OIHW', 'NCHW'))
    return y + b[None, :, None, None]


def _block(x, p):
    x = jax.nn.relu(_conv(x, p['w1'], p['b1'], 1, 1))
    x = jax.nn.relu(_conv(x, p['w2'], p['b2'], 1, 1))
    return x


def _maxpool(x):
    return lax.reduce_window(x, -jnp.inf, lax.max,
                             (1, 1, 2, 2), (1, 1, 2, 2), 'VALID')


def _argmin_kernel(n_tok, z_ref, cb_ref, idx_ref, loss_ref):
    z = z_ref[...]
    cb = cb_ref[...]
    pad = z.shape[0]
    zn = jnp.sum(z * z, axis=1, keepdims=True)
    cn = jnp.sum(cb * cb, axis=1)[None, :]
    dot = lax.dot_general(z, cb, (((1,), (1,)), ((), ())),
                          preferred_element_type=jnp.float32)
    d2 = (zn + cn) - 2.0 * dot
    d2 = jnp.maximum(d2, 0.0)
    mn = jnp.min(d2, axis=1, keepdims=True)
    lane = lax.broadcasted_iota(jnp.int32, d2.shape, 1)
    idx = jnp.min(jnp.where(d2 == mn, lane, _NUM_EMB), axis=1)
    idx_ref[...] = idx.reshape(1, pad)
    mask = (lax.broadcasted_iota(jnp.int32, (pad, 1), 0) < n_tok
            ).astype(jnp.float32)
    m = jnp.sum(mn * mask, axis=0, keepdims=True) / (n_tok * _EMBED_DIM)
    loss_ref[...] = m + _BETA * m


_SC_CORES = 1
_SC_SUBCORES = 16
_NW = _SC_CORES * _SC_SUBCORES
_BPAD = 8 * _NW
_BPW = _BPAD // _NW


def _sc_gather_body(cb_hbm, idx_hbm, out_hbm, idx_v, rows_v, sem):
    wid = lax.axis_index("s") * _SC_CORES + lax.axis_index("c")
    base = wid * _BPW
    pltpu.sync_copy(idx_hbm.at[pl.ds(base, _BPW)], idx_v)
    pltpu.async_copy(cb_hbm.at[idx_v], rows_v, sem).wait()
    pltpu.sync_copy(rows_v, out_hbm.at[pl.ds(base, _BPW)])


_sc_gather = functools.partial(
    pl.kernel,
    out_type=jax.ShapeDtypeStruct((_BPAD, _EMBED_DIM), jnp.float32),
    mesh=plsc.VectorSubcoreMesh(core_axis_name="c", subcore_axis_name="s",
                                num_cores=_SC_CORES,
                                num_subcores=_SC_SUBCORES),
    compiler_params=pltpu.CompilerParams(use_tc_tiling_on_sc=False),
    scratch_types=[
        pltpu.VMEM((_BPW,), jnp.int32),
        pltpu.VMEM((_BPW, _EMBED_DIM), jnp.float32),
        pltpu.SemaphoreType.DMA,
    ],
)(_sc_gather_body)


def _vq(z_flat, cb):
    n_tok = z_flat.shape[0]
    padded = max(_BPAD, -(-n_tok // 8) * 8)
    zp = jnp.pad(z_flat, ((0, padded - n_tok), (0, 0)))
    idx, loss = pl.pallas_call(
        functools.partial(_argmin_kernel, n_tok),
        out_shape=(jax.ShapeDtypeStruct((1, padded), jnp.int32),
                   jax.ShapeDtypeStruct((1, 1), jnp.float32)),
    )(zp, cb)
    zq_pad = _sc_gather(cb, idx.reshape(padded))
    return zq_pad[:n_tok], loss[0, 0]


def kernel(x, params):
    h = x
    for p in params['enc']:
        h = _maxpool(_block(h, p))
    z_e = _conv(h, params['pre_w'], params['pre_b'], 1, 0)
    B, C, H, W = z_e.shape
    z_e_flat = jnp.transpose(z_e, (0, 2, 3, 1)).reshape(B * H * W, C)
    z_q, q_loss = _vq(z_e_flat, params['codebook'])
    latent = jnp.transpose(z_q.reshape(B, H, W, C), (0, 3, 1, 2))
    z = _conv(latent, params['post_w'], params['post_b'], 1, 0)
    for p in params['dec']:
        z = _block(_convT(z, p['wt'], p['bt']), p)
    x_reconst = jnp.tanh(_convT(z, params['dec_final_w'],
                                params['dec_final_b']))
    return (x_reconst, latent, q_loss)
